# baseline (speedup 1.0000x reference)
import functools

import jax
import jax.numpy as jnp
from jax import lax
from jax.experimental import pallas as pl
from jax.experimental.pallas import tpu as pltpu
from jax.experimental.pallas import tpu_sc as plsc

LANES = 16
NW = 32
LP = 56
G = 2
NBUF = 4


def _sc_pool_body(RPW, idx_hbm, table_hbm, out_hbm, idx_v, out_v, bufs, sems):
    D = table_hbm.shape[1]
    DK = D // LANES
    wid = lax.axis_index("s") * 2 + lax.axis_index("c")
    base = wid * RPW

    pltpu.sync_copy(idx_hbm.at[pl.ds(base * LP, RPW * LP)], idx_v)

    NG = RPW // G

    def _copy(g, slot):
        return pltpu.make_async_copy(
            table_hbm.at[idx_v.at[pl.ds(g * (G * LP), G * LP)]],
            bufs[slot], sems[slot])

    def _accum(g, slot):
        buf = bufs[slot]
        for rr in range(G):
            r = g * G + rr

            def tok_body(j, accs):
                return tuple(
                    accs[k] + buf[rr * LP + j, pl.ds(k * LANES, LANES)]
                    for k in range(DK))

            accs = lax.fori_loop(
                0, LP, tok_body,
                tuple(jnp.zeros((LANES,), jnp.float32) for _ in range(DK)),
                unroll=4)
            for k in range(DK):
                out_v[r, pl.ds(k * LANES, LANES)] = accs[k]

    for slot in range(NBUF):
        _copy(slot, slot).start()

    def outer(g0, _):
        for slot in range(NBUF):
            g = g0 + slot
            _copy(g, slot).wait()
            _accum(g, slot)

            @pl.when(g + NBUF < NG)
            def _():
                _copy(g + NBUF, slot).start()
        return 0

    lax.fori_loop(0, NG // NBUF, lambda i, c: outer(i * NBUF, c), 0)

    pltpu.sync_copy(out_v, out_hbm.at[pl.ds(base, RPW), :])


def _sc_pool(idx_flat, table, B):
    D = table.shape[1]
    RPW = B // NW
    mesh = plsc.VectorSubcoreMesh(core_axis_name="c", subcore_axis_name="s")
    f = pl.kernel(
        functools.partial(_sc_pool_body, RPW),
        out_type=jax.ShapeDtypeStruct((B, D), jnp.float32),
        mesh=mesh,
        scratch_types=dict(
            idx_v=pltpu.VMEM((RPW * LP,), jnp.int32),
            out_v=pltpu.VMEM((RPW, D), jnp.float32),
            bufs=[pltpu.VMEM((G * LP, D), jnp.float32) for _ in range(NBUF)],
            sems=[pltpu.SemaphoreType.DMA for _ in range(NBUF)],
        ),
    )
    return f(idx_flat, table)


def _mm_body(s_ref, m_ref, t0_ref, w_ref, b_ref, o_ref):
    m = m_ref[...].astype(jnp.float32)
    pooled = (s_ref[...] - (LP - m) * t0_ref[...]) / m
    o_ref[...] = jnp.dot(pooled, w_ref[...],
                         preferred_element_type=jnp.float32) + b_ref[...]


def _tc_transform(sums, m, table0, W, b):
    B, D = sums.shape
    E = W.shape[1]
    BM = 512
    return pl.pallas_call(
        _mm_body,
        grid=(B // BM,),
        in_specs=[
            pl.BlockSpec((BM, D), lambda i: (i, 0)),
            pl.BlockSpec((BM, 1), lambda i: (i, 0)),
            pl.BlockSpec((1, D), lambda i: (0, 0)),
            pl.BlockSpec((D, E), lambda i: (0, 0)),
            pl.BlockSpec((1, E), lambda i: (0, 0)),
        ],
        out_specs=pl.BlockSpec((BM, E), lambda i: (i, 0)),
        out_shape=jax.ShapeDtypeStruct((B, E), jnp.float32),
    )(sums, m.reshape(B, 1), table0, W, b.reshape(1, E))


def kernel(token_indices, seq_lengths, table, W, b):
    B, L = token_indices.shape
    m = jnp.maximum(seq_lengths, 1)
    idx_pad = jnp.pad(token_indices, ((0, 0), (0, LP - L)))
    idx_clean = jnp.where(jnp.arange(LP)[None, :] < m[:, None], idx_pad, 0)
    sums = _sc_pool(idx_clean.reshape(-1), table, B)
    return _tc_transform(sums, m, table[0:1, :], W, b)
